# baseline (speedup 1.0000x reference)
import jax
import jax.numpy as jnp
from jax.experimental import pallas as pl
from jax.experimental.pallas import tpu as pltpu

B = 16
N = 4096
D = 256
XBLK = 1024
NSTEP = N // XBLK


def _graph_kernel(x_ref, starts_ref, ends_ref, wb_ref, bb_ref, wp_ref,
                  bp_ref, wr_ref, br_ref, ap_ref, wqp_ref, ar_ref, wqr_ref,
                  keys_ref, pq_ref, rq_ref, featsb_ref):
    bf16 = jnp.bfloat16
    pid = pl.program_id(0)

    x = x_ref[...].astype(bf16)
    fc = jnp.maximum(jnp.dot(x, wb_ref[...].astype(bf16),
                             preferred_element_type=jnp.float32)
                     + bb_ref[...], 0.0)
    featsb_ref[pid] = fc.astype(bf16)

    @pl.when(pid == NSTEP - 1)
    def _tail():
        featsb = featsb_ref[...].reshape(N, D)

        ids = jax.lax.broadcasted_iota(jnp.int32, (B, N), 1)
        starts = starts_ref[...]
        ends = ends_ref[...]
        seg = jnp.logical_and(ids >= starts, ids < ends)

        wpr = jnp.concatenate([wp_ref[...], wr_ref[...]],
                              axis=1).astype(bf16)
        bpr = jnp.concatenate([bp_ref[...], br_ref[...]], axis=1)
        feat2 = jnp.maximum(jnp.dot(featsb, wpr,
                                    preferred_element_type=jnp.float32)
                            + bpr, 0.0)
        feat2b = feat2.astype(bf16)

        def weights(sl, att_ref):
            scores = jax.lax.dot_general(
                att_ref[...].astype(bf16), feat2b[:, sl],
                (((1,), (1,)), ((), ())),
                preferred_element_type=jnp.float32)
            e_row = jnp.exp(scores - jnp.max(scores))
            e2d = jnp.where(seg, e_row, 0.0)
            l = jnp.sum(e2d, axis=1, keepdims=True)
            return e2d.astype(bf16), 1.0 / jnp.maximum(l, 1e-30)

        slp, slr = slice(0, D), slice(D, 2 * D)
        ep, ilp = weights(slp, ap_ref)
        er, ilr = weights(slr, ar_ref)

        seg_sum = jnp.dot(seg.astype(bf16), featsb,
                          preferred_element_type=jnp.float32)
        inv_len = 1.0 / jnp.maximum((ends - starts).astype(jnp.float32),
                                    1.0)
        keys_ref[...] = seg_sum * inv_len

        pooled_p = jnp.dot(ep, feat2b[:, slp],
                           preferred_element_type=jnp.float32) * ilp
        pooled_r = jnp.dot(er, feat2b[:, slr],
                           preferred_element_type=jnp.float32) * ilr
        pq_ref[...] = jnp.dot(pooled_p, wqp_ref[...],
                              preferred_element_type=jnp.float32)
        rq_ref[...] = jnp.dot(pooled_r, wqr_ref[...],
                              preferred_element_type=jnp.float32)


def kernel(x, cu_seqlens, W_base, b_base, W_p, b_p, W_r, b_r,
           w_att_p, W_q_p, w_att_r, W_q_r):
    cu = cu_seqlens.astype(jnp.int32)
    starts = cu[:-1].reshape(B, 1)
    ends = cu[1:].reshape(B, 1)
    full = lambda shape: pl.BlockSpec(shape, lambda i: (0,) * len(shape))
    in_specs = [
        pl.BlockSpec((XBLK, D), lambda i: (i, 0)),
        full((B, 1)), full((B, 1)),
        full((D, D)), full((1, D)),
        full((D, D)), full((1, D)),
        full((D, D)), full((1, D)),
        full((1, D)), full((D, D)),
        full((1, D)), full((D, D)),
    ]
    out_specs = (full((B, D)), full((B, D)), full((B, D)))
    out_shape = tuple(jax.ShapeDtypeStruct((B, D), jnp.float32)
                      for _ in range(3))
    return pl.pallas_call(
        _graph_kernel,
        grid=(NSTEP,),
        in_specs=in_specs,
        out_specs=out_specs,
        out_shape=out_shape,
        scratch_shapes=[pltpu.VMEM((NSTEP, XBLK, D), jnp.bfloat16)],
    )(x, starts, ends,
      W_base, b_base.reshape(1, D),
      W_p, b_p.reshape(1, D),
      W_r, b_r.reshape(1, D),
      w_att_p.reshape(1, D), W_q_p,
      w_att_r.reshape(1, D), W_q_r)


# speedup vs baseline: 1.1118x; 1.1118x over previous
import jax
import jax.numpy as jnp
from jax.experimental import pallas as pl

B = 16
N = 4096
D = 256


def _graph_kernel(x_ref, starts_ref, ends_ref, wb_ref, bb_ref, wp_ref,
                  bp_ref, wr_ref, br_ref, ap_ref, wqp_ref, ar_ref, wqr_ref,
                  keys_ref, pq_ref, rq_ref):
    bf16 = jnp.bfloat16
    x = x_ref[...].astype(bf16)
    feats = jnp.maximum(jnp.dot(x, wb_ref[...].astype(bf16),
                                preferred_element_type=jnp.float32)
                        + bb_ref[...], 0.0)
    featsb = feats.astype(bf16)

    ids = jax.lax.broadcasted_iota(jnp.int32, (B, N), 1)
    starts = starts_ref[...]
    ends = ends_ref[...]
    seg = jnp.logical_and(ids >= starts, ids < ends)

    wpr = jnp.concatenate([wp_ref[...], wr_ref[...]], axis=1).astype(bf16)
    bpr = jnp.concatenate([bp_ref[...], br_ref[...]], axis=1)
    feat2 = jnp.maximum(jnp.dot(featsb, wpr,
                                preferred_element_type=jnp.float32)
                        + bpr, 0.0)
    feat2b = feat2.astype(bf16)

    def weights(sl, att_ref):
        scores = jax.lax.dot_general(
            att_ref[...].astype(bf16), feat2b[:, sl],
            (((1,), (1,)), ((), ())),
            preferred_element_type=jnp.float32)
        e_row = jnp.exp(scores - jnp.max(scores))
        e2d = jnp.where(seg, e_row, 0.0)
        l = jnp.sum(e2d, axis=1, keepdims=True)
        return e2d.astype(bf16), 1.0 / jnp.maximum(l, 1e-30)

    slp, slr = slice(0, D), slice(D, 2 * D)
    ep, ilp = weights(slp, ap_ref)
    er, ilr = weights(slr, ar_ref)

    seg_sum = jnp.dot(seg.astype(bf16), featsb,
                      preferred_element_type=jnp.float32)
    inv_len = 1.0 / jnp.maximum((ends - starts).astype(jnp.float32), 1.0)
    keys_ref[...] = seg_sum * inv_len

    pooled_p = jnp.dot(ep, feat2b[:, slp],
                       preferred_element_type=jnp.float32) * ilp
    pooled_r = jnp.dot(er, feat2b[:, slr],
                       preferred_element_type=jnp.float32) * ilr
    pq_ref[...] = jnp.dot(pooled_p, wqp_ref[...],
                          preferred_element_type=jnp.float32)
    rq_ref[...] = jnp.dot(pooled_r, wqr_ref[...],
                          preferred_element_type=jnp.float32)


def kernel(x, cu_seqlens, W_base, b_base, W_p, b_p, W_r, b_r,
           w_att_p, W_q_p, w_att_r, W_q_r):
    cu = cu_seqlens.astype(jnp.int32)
    starts = cu[:-1].reshape(B, 1)
    ends = cu[1:].reshape(B, 1)
    out_shape = tuple(jax.ShapeDtypeStruct((B, D), jnp.float32)
                      for _ in range(3))
    return pl.pallas_call(
        _graph_kernel,
        out_shape=out_shape,
    )(x, starts, ends,
      W_base, b_base.reshape(1, D),
      W_p, b_p.reshape(1, D),
      W_r, b_r.reshape(1, D),
      w_att_p.reshape(1, D), W_q_p,
      w_att_r.reshape(1, D), W_q_r)
